# baseline (speedup 1.0000x reference)
import jax
import jax.numpy as jnp
from jax.experimental import pallas as pl
from jax.experimental.pallas import tpu as pltpu

B, N, H, W = 16, 512, 128, 128
HW = H * W
D_ENT_IN, D_EMB = 128, 256
D_SCATTER = 32
C_SPATIAL = 64
PIX = 2048
ROWS = PIX // W
NBLK = HW // PIX


def _entity_kernel(num_ref, ef_ref, went_ref, wproj_ref, w1_ref,
                   ee_ref, ment_ref, q_ref):
    b = pl.program_id(0)
    ef = ef_ref[0]
    ee = jax.nn.relu(jnp.dot(ef, went_ref[...],
                             preferred_element_type=jnp.float32))
    ee_ref[0] = ee
    num = num_ref[b]
    iota = jax.lax.broadcasted_iota(jnp.int32, (N, 1), 0)
    maskf = (iota < num).astype(jnp.float32)
    denom = jnp.maximum(num, 1).astype(jnp.float32)
    ment_ref[0, 0] = (ee * maskf).sum(axis=0) / denom
    proj = jax.nn.relu(jnp.dot(ee, wproj_ref[...],
                               preferred_element_type=jnp.float32)) * maskf
    q_ref[0] = jnp.dot(proj, w1_ref[...], preferred_element_type=jnp.float32)


def _spatial_kernel(x_ref, y_ref, h_ref, q_ref, w0_ref,
                    map_ref, pool_ref):
    j = pl.program_id(1)
    fidx = y_ref[0] * W + x_ref[0]
    pix = (jax.lax.broadcasted_iota(jnp.int32, (PIX, N), 0) + j * PIX)
    onehot = (pix == fidx).astype(jnp.bfloat16)
    qb = q_ref[0].astype(jnp.bfloat16)
    acc = jnp.dot(onehot, qb, preferred_element_type=jnp.float32)
    out = jax.nn.relu(h_ref[...] * w0_ref[...] + acc)
    map_ref[...] = out.reshape(1, ROWS, W, C_SPATIAL)
    psum = out.sum(axis=0, keepdims=True)

    @pl.when(j == 0)
    def _():
        pool_ref[0] = psum

    @pl.when(j > 0)
    def _():
        pool_ref[0] += psum


def _head_kernel(sf_ref, wsc_ref, wctx_ref, wbase_ref,
                 ment_ref, wee_ref, pool_ref, wsp_ref,
                 lstm_ref, ctx_ref, base_ref):
    es = jax.nn.relu(jnp.dot(sf_ref[...], wsc_ref[...],
                             preferred_element_type=jnp.float32))
    ctx_ref[...] = jax.nn.relu(jnp.dot(es, wctx_ref[...],
                                       preferred_element_type=jnp.float32))
    base_ref[...] = jax.nn.relu(jnp.dot(es, wbase_ref[...],
                                        preferred_element_type=jnp.float32))
    eent = jax.nn.relu(jnp.dot(ment_ref[...], wee_ref[...],
                               preferred_element_type=jnp.float32))
    pooled = pool_ref[...] / float(HW)
    esp = jax.nn.relu(jnp.dot(pooled, wsp_ref[...],
                              preferred_element_type=jnp.float32))
    lstm_ref[:, 0:256] = es
    lstm_ref[:, 256:512] = eent
    lstm_ref[:, 512:768] = esp


def kernel(spatial_height_map, entity_features, scalar_features, entity_x,
           entity_y, entity_num, W_scalar, W_ctx, W_base, W_ent, W_ent_emb,
           W_proj, W_spatial, W_sp_emb):
    x3 = entity_x.astype(jnp.int32).reshape(B, 1, N)
    y3 = entity_y.astype(jnp.int32).reshape(B, 1, N)
    num = entity_num.astype(jnp.int32)
    w0 = W_spatial[0:1, :]
    w1 = W_spatial[1:, :]
    hflat = spatial_height_map.reshape(B * HW, 1)

    ee, ment, q = pl.pallas_call(
        _entity_kernel,
        grid_spec=pltpu.PrefetchScalarGridSpec(
            num_scalar_prefetch=1,
            grid=(B,),
            in_specs=[
                pl.BlockSpec((1, N, D_ENT_IN), lambda b, *_: (b, 0, 0)),
                pl.BlockSpec((D_ENT_IN, D_EMB), lambda b, *_: (0, 0)),
                pl.BlockSpec((D_EMB, D_SCATTER), lambda b, *_: (0, 0)),
                pl.BlockSpec((D_SCATTER, C_SPATIAL), lambda b, *_: (0, 0)),
            ],
            out_specs=[
                pl.BlockSpec((1, N, D_EMB), lambda b, *_: (b, 0, 0)),
                pl.BlockSpec((1, 1, D_EMB), lambda b, *_: (b, 0, 0)),
                pl.BlockSpec((1, N, C_SPATIAL), lambda b, *_: (b, 0, 0)),
            ],
        ),
        out_shape=[
            jax.ShapeDtypeStruct((B, N, D_EMB), jnp.float32),
            jax.ShapeDtypeStruct((B, 1, D_EMB), jnp.float32),
            jax.ShapeDtypeStruct((B, N, C_SPATIAL), jnp.float32),
        ],
    )(num, entity_features, W_ent, W_proj, w1)

    map_skip, pool_sum = pl.pallas_call(
        _spatial_kernel,
        grid=(B, NBLK),
        in_specs=[
            pl.BlockSpec((1, 1, N), lambda b, j: (b, 0, 0)),
            pl.BlockSpec((1, 1, N), lambda b, j: (b, 0, 0)),
            pl.BlockSpec((PIX, 1), lambda b, j: (b * NBLK + j, 0)),
            pl.BlockSpec((1, N, C_SPATIAL), lambda b, j: (b, 0, 0)),
            pl.BlockSpec((1, C_SPATIAL), lambda b, j: (0, 0)),
        ],
        out_specs=[
            pl.BlockSpec((1, ROWS, W, C_SPATIAL), lambda b, j: (b, j, 0, 0)),
            pl.BlockSpec((1, 1, C_SPATIAL), lambda b, j: (b, 0, 0)),
        ],
        out_shape=[
            jax.ShapeDtypeStruct((B, H, W, C_SPATIAL), jnp.float32),
            jax.ShapeDtypeStruct((B, 1, C_SPATIAL), jnp.float32),
        ],
    )(x3, y3, hflat, q, w0)

    lstm_input, scalar_context, baseline_feature = pl.pallas_call(
        _head_kernel,
        in_specs=[
            pl.BlockSpec((B, 256), lambda: (0, 0)),
            pl.BlockSpec((256, 256), lambda: (0, 0)),
            pl.BlockSpec((256, 128), lambda: (0, 0)),
            pl.BlockSpec((256, 64), lambda: (0, 0)),
            pl.BlockSpec((B, D_EMB), lambda: (0, 0)),
            pl.BlockSpec((D_EMB, D_EMB), lambda: (0, 0)),
            pl.BlockSpec((B, C_SPATIAL), lambda: (0, 0)),
            pl.BlockSpec((C_SPATIAL, 256), lambda: (0, 0)),
        ],
        out_specs=[
            pl.BlockSpec((B, 768), lambda: (0, 0)),
            pl.BlockSpec((B, 128), lambda: (0, 0)),
            pl.BlockSpec((B, 64), lambda: (0, 0)),
        ],
        out_shape=[
            jax.ShapeDtypeStruct((B, 768), jnp.float32),
            jax.ShapeDtypeStruct((B, 128), jnp.float32),
            jax.ShapeDtypeStruct((B, 64), jnp.float32),
        ],
    )(scalar_features, W_scalar, W_ctx, W_base,
      ment.reshape(B, D_EMB), W_ent_emb, pool_sum.reshape(B, C_SPATIAL),
      W_sp_emb)

    return (lstm_input, scalar_context, baseline_feature, ee, map_skip)


# speedup vs baseline: 1.3236x; 1.3236x over previous
import jax
import jax.numpy as jnp
from jax.experimental import pallas as pl
from jax.experimental.pallas import tpu as pltpu

B, N, H, W = 16, 512, 128, 128
HW = H * W
D_ENT_IN, D_EMB = 128, 256
D_SCATTER = 32
C_SPATIAL = 64
PIX = 2048
ROWS = PIX // W
NBLK = HW // PIX


def _entity_kernel(num_ref, ef_ref, went_ref, wproj_ref, w1_ref,
                   ee_ref, ment_ref, q_ref):
    b = pl.program_id(0)
    ef = ef_ref[0]
    ee = jax.nn.relu(jnp.dot(ef, went_ref[...],
                             preferred_element_type=jnp.float32))
    ee_ref[0] = ee
    num = num_ref[b]
    iota = jax.lax.broadcasted_iota(jnp.int32, (N, 1), 0)
    maskf = (iota < num).astype(jnp.float32)
    denom = jnp.maximum(num, 1).astype(jnp.float32)
    ment_ref[0, 0] = (ee * maskf).sum(axis=0) / denom
    proj = jax.nn.relu(jnp.dot(ee, wproj_ref[...],
                               preferred_element_type=jnp.float32)) * maskf
    q_ref[0] = jnp.dot(proj, w1_ref[...], preferred_element_type=jnp.float32)


def _spatial_kernel(idx_ref, h_ref, q_ref, w0_ref, map_ref, pool_ref):
    b = pl.program_id(0)
    map_ref[...] = h_ref[...] * w0_ref[...]

    def body(e, carry):
        i = idx_ref[b, e]
        row = map_ref[pl.ds(i, 1), :]
        map_ref[pl.ds(i, 1), :] = row + q_ref[0, pl.ds(e, 1), :]
        return carry

    jax.lax.fori_loop(0, N, body, 0, unroll=8)
    out = jax.nn.relu(map_ref[...])
    map_ref[...] = out
    pool_ref[...] = out.sum(axis=0).reshape(1, 1, C_SPATIAL)


def _head_kernel(sf_ref, wsc_ref, wctx_ref, wbase_ref,
                 ment_ref, wee_ref, pool_ref, wsp_ref,
                 lstm_ref, ctx_ref, base_ref):
    es = jax.nn.relu(jnp.dot(sf_ref[...], wsc_ref[...],
                             preferred_element_type=jnp.float32))
    ctx_ref[...] = jax.nn.relu(jnp.dot(es, wctx_ref[...],
                                       preferred_element_type=jnp.float32))
    base_ref[...] = jax.nn.relu(jnp.dot(es, wbase_ref[...],
                                        preferred_element_type=jnp.float32))
    eent = jax.nn.relu(jnp.dot(ment_ref[...], wee_ref[...],
                               preferred_element_type=jnp.float32))
    pooled = pool_ref[...] / float(HW)
    esp = jax.nn.relu(jnp.dot(pooled, wsp_ref[...],
                              preferred_element_type=jnp.float32))
    lstm_ref[:, 0:256] = es
    lstm_ref[:, 256:512] = eent
    lstm_ref[:, 512:768] = esp


def kernel(spatial_height_map, entity_features, scalar_features, entity_x,
           entity_y, entity_num, W_scalar, W_ctx, W_base, W_ent, W_ent_emb,
           W_proj, W_spatial, W_sp_emb):
    x3 = entity_x.astype(jnp.int32).reshape(B, 1, N)
    y3 = entity_y.astype(jnp.int32).reshape(B, 1, N)
    num = entity_num.astype(jnp.int32)
    w0 = W_spatial[0:1, :]
    w1 = W_spatial[1:, :]
    hflat = spatial_height_map.reshape(B * HW, 1)

    ee, ment, q = pl.pallas_call(
        _entity_kernel,
        grid_spec=pltpu.PrefetchScalarGridSpec(
            num_scalar_prefetch=1,
            grid=(B,),
            in_specs=[
                pl.BlockSpec((1, N, D_ENT_IN), lambda b, *_: (b, 0, 0)),
                pl.BlockSpec((D_ENT_IN, D_EMB), lambda b, *_: (0, 0)),
                pl.BlockSpec((D_EMB, D_SCATTER), lambda b, *_: (0, 0)),
                pl.BlockSpec((D_SCATTER, C_SPATIAL), lambda b, *_: (0, 0)),
            ],
            out_specs=[
                pl.BlockSpec((1, N, D_EMB), lambda b, *_: (b, 0, 0)),
                pl.BlockSpec((1, 1, D_EMB), lambda b, *_: (b, 0, 0)),
                pl.BlockSpec((1, N, C_SPATIAL), lambda b, *_: (b, 0, 0)),
            ],
        ),
        out_shape=[
            jax.ShapeDtypeStruct((B, N, D_EMB), jnp.float32),
            jax.ShapeDtypeStruct((B, 1, D_EMB), jnp.float32),
            jax.ShapeDtypeStruct((B, N, C_SPATIAL), jnp.float32),
        ],
    )(num, entity_features, W_ent, W_proj, w1)

    fidx = (y3 * W + x3).reshape(B, N)

    map_flat, pool_sum = pl.pallas_call(
        _spatial_kernel,
        grid_spec=pltpu.PrefetchScalarGridSpec(
            num_scalar_prefetch=1,
            grid=(B,),
            in_specs=[
                pl.BlockSpec((HW, 1), lambda b, *_: (b, 0)),
                pl.BlockSpec((1, N, C_SPATIAL), lambda b, *_: (b, 0, 0)),
                pl.BlockSpec((1, C_SPATIAL), lambda b, *_: (0, 0)),
            ],
            out_specs=[
                pl.BlockSpec((HW, C_SPATIAL), lambda b, *_: (b, 0)),
                pl.BlockSpec((1, 1, C_SPATIAL), lambda b, *_: (b, 0, 0)),
            ],
        ),
        out_shape=[
            jax.ShapeDtypeStruct((B * HW, C_SPATIAL), jnp.float32),
            jax.ShapeDtypeStruct((B, 1, C_SPATIAL), jnp.float32),
        ],
    )(fidx, hflat, q, w0)
    map_skip = map_flat.reshape(B, H, W, C_SPATIAL)

    lstm_input, scalar_context, baseline_feature = pl.pallas_call(
        _head_kernel,
        in_specs=[
            pl.BlockSpec((B, 256), lambda: (0, 0)),
            pl.BlockSpec((256, 256), lambda: (0, 0)),
            pl.BlockSpec((256, 128), lambda: (0, 0)),
            pl.BlockSpec((256, 64), lambda: (0, 0)),
            pl.BlockSpec((B, D_EMB), lambda: (0, 0)),
            pl.BlockSpec((D_EMB, D_EMB), lambda: (0, 0)),
            pl.BlockSpec((B, C_SPATIAL), lambda: (0, 0)),
            pl.BlockSpec((C_SPATIAL, 256), lambda: (0, 0)),
        ],
        out_specs=[
            pl.BlockSpec((B, 768), lambda: (0, 0)),
            pl.BlockSpec((B, 128), lambda: (0, 0)),
            pl.BlockSpec((B, 64), lambda: (0, 0)),
        ],
        out_shape=[
            jax.ShapeDtypeStruct((B, 768), jnp.float32),
            jax.ShapeDtypeStruct((B, 128), jnp.float32),
            jax.ShapeDtypeStruct((B, 64), jnp.float32),
        ],
    )(scalar_features, W_scalar, W_ctx, W_base,
      ment.reshape(B, D_EMB), W_ent_emb, pool_sum.reshape(B, C_SPATIAL),
      W_sp_emb)

    return (lstm_input, scalar_context, baseline_feature, ee, map_skip)


# speedup vs baseline: 1.6808x; 1.2699x over previous
import jax
import jax.numpy as jnp
from jax.experimental import pallas as pl
from jax.experimental.pallas import tpu as pltpu

B, N, H, W = 16, 512, 128, 128
HW = H * W
D_ENT_IN, D_EMB = 128, 256
D_SCATTER = 32
C_SPATIAL = 64
PIX = 2048
ROWS = PIX // W
NBLK = HW // PIX


def _entity_kernel(num_ref, ef_ref, went_ref, wproj_ref, w1_ref,
                   ee_ref, ment_ref, q_ref):
    b = pl.program_id(0)
    ef = ef_ref[0]
    ee = jax.nn.relu(jnp.dot(ef, went_ref[...],
                             preferred_element_type=jnp.float32))
    ee_ref[0] = ee
    num = num_ref[b]
    iota = jax.lax.broadcasted_iota(jnp.int32, (N, 1), 0)
    maskf = (iota < num).astype(jnp.float32)
    denom = jnp.maximum(num, 1).astype(jnp.float32)
    ment_ref[0, 0] = (ee * maskf).sum(axis=0) / denom
    proj = jax.nn.relu(jnp.dot(ee, wproj_ref[...],
                               preferred_element_type=jnp.float32)) * maskf
    q_ref[0] = jnp.dot(proj, w1_ref[...], preferred_element_type=jnp.float32)


def _spatial_kernel(idx_ref, h_ref, q_ref, w0_ref, map_ref, pool_ref):
    b = pl.program_id(0)
    h2t = h_ref[0].T
    w0v = w0_ref[...]
    for y in range(H):
        col = jax.lax.slice(h2t, (0, y), (W, y + 1))
        map_ref[y * W:(y + 1) * W, :] = col * w0v

    def body(e, carry):
        i = idx_ref[b, e]
        row = map_ref[pl.ds(i, 1), :]
        map_ref[pl.ds(i, 1), :] = row + q_ref[0, pl.ds(e, 1), :]
        return carry

    jax.lax.fori_loop(0, N, body, 0, unroll=8)
    out = jax.nn.relu(map_ref[...])
    map_ref[...] = out
    pool_ref[...] = out.sum(axis=0).reshape(1, 1, C_SPATIAL)


def _head_kernel(sf_ref, wsc_ref, wctx_ref, wbase_ref,
                 ment_ref, wee_ref, pool_ref, wsp_ref,
                 lstm_ref, ctx_ref, base_ref):
    es = jax.nn.relu(jnp.dot(sf_ref[...], wsc_ref[...],
                             preferred_element_type=jnp.float32))
    ctx_ref[...] = jax.nn.relu(jnp.dot(es, wctx_ref[...],
                                       preferred_element_type=jnp.float32))
    base_ref[...] = jax.nn.relu(jnp.dot(es, wbase_ref[...],
                                        preferred_element_type=jnp.float32))
    eent = jax.nn.relu(jnp.dot(ment_ref[...], wee_ref[...],
                               preferred_element_type=jnp.float32))
    pooled = pool_ref[...] / float(HW)
    esp = jax.nn.relu(jnp.dot(pooled, wsp_ref[...],
                              preferred_element_type=jnp.float32))
    lstm_ref[:, 0:256] = es
    lstm_ref[:, 256:512] = eent
    lstm_ref[:, 512:768] = esp


def kernel(spatial_height_map, entity_features, scalar_features, entity_x,
           entity_y, entity_num, W_scalar, W_ctx, W_base, W_ent, W_ent_emb,
           W_proj, W_spatial, W_sp_emb):
    x3 = entity_x.astype(jnp.int32).reshape(B, 1, N)
    y3 = entity_y.astype(jnp.int32).reshape(B, 1, N)
    num = entity_num.astype(jnp.int32)
    w0 = W_spatial[0:1, :]
    w1 = W_spatial[1:, :]
    hflat = spatial_height_map.reshape(B * HW, 1)

    ee, ment, q = pl.pallas_call(
        _entity_kernel,
        grid_spec=pltpu.PrefetchScalarGridSpec(
            num_scalar_prefetch=1,
            grid=(B,),
            in_specs=[
                pl.BlockSpec((1, N, D_ENT_IN), lambda b, *_: (b, 0, 0)),
                pl.BlockSpec((D_ENT_IN, D_EMB), lambda b, *_: (0, 0)),
                pl.BlockSpec((D_EMB, D_SCATTER), lambda b, *_: (0, 0)),
                pl.BlockSpec((D_SCATTER, C_SPATIAL), lambda b, *_: (0, 0)),
            ],
            out_specs=[
                pl.BlockSpec((1, N, D_EMB), lambda b, *_: (b, 0, 0)),
                pl.BlockSpec((1, 1, D_EMB), lambda b, *_: (b, 0, 0)),
                pl.BlockSpec((1, N, C_SPATIAL), lambda b, *_: (b, 0, 0)),
            ],
        ),
        out_shape=[
            jax.ShapeDtypeStruct((B, N, D_EMB), jnp.float32),
            jax.ShapeDtypeStruct((B, 1, D_EMB), jnp.float32),
            jax.ShapeDtypeStruct((B, N, C_SPATIAL), jnp.float32),
        ],
    )(num, entity_features, W_ent, W_proj, w1)

    fidx = (y3 * W + x3).reshape(B, N)

    map_flat, pool_sum = pl.pallas_call(
        _spatial_kernel,
        grid_spec=pltpu.PrefetchScalarGridSpec(
            num_scalar_prefetch=1,
            grid=(B,),
            in_specs=[
                pl.BlockSpec((1, H, W), lambda b, *_: (b, 0, 0)),
                pl.BlockSpec((1, N, C_SPATIAL), lambda b, *_: (b, 0, 0)),
                pl.BlockSpec((1, C_SPATIAL), lambda b, *_: (0, 0)),
            ],
            out_specs=[
                pl.BlockSpec((HW, C_SPATIAL), lambda b, *_: (b, 0)),
                pl.BlockSpec((1, 1, C_SPATIAL), lambda b, *_: (b, 0, 0)),
            ],
        ),
        out_shape=[
            jax.ShapeDtypeStruct((B * HW, C_SPATIAL), jnp.float32),
            jax.ShapeDtypeStruct((B, 1, C_SPATIAL), jnp.float32),
        ],
    )(fidx, spatial_height_map, q, w0)
    map_skip = map_flat.reshape(B, H, W, C_SPATIAL)

    lstm_input, scalar_context, baseline_feature = pl.pallas_call(
        _head_kernel,
        in_specs=[
            pl.BlockSpec((B, 256), lambda: (0, 0)),
            pl.BlockSpec((256, 256), lambda: (0, 0)),
            pl.BlockSpec((256, 128), lambda: (0, 0)),
            pl.BlockSpec((256, 64), lambda: (0, 0)),
            pl.BlockSpec((B, D_EMB), lambda: (0, 0)),
            pl.BlockSpec((D_EMB, D_EMB), lambda: (0, 0)),
            pl.BlockSpec((B, C_SPATIAL), lambda: (0, 0)),
            pl.BlockSpec((C_SPATIAL, 256), lambda: (0, 0)),
        ],
        out_specs=[
            pl.BlockSpec((B, 768), lambda: (0, 0)),
            pl.BlockSpec((B, 128), lambda: (0, 0)),
            pl.BlockSpec((B, 64), lambda: (0, 0)),
        ],
        out_shape=[
            jax.ShapeDtypeStruct((B, 768), jnp.float32),
            jax.ShapeDtypeStruct((B, 128), jnp.float32),
            jax.ShapeDtypeStruct((B, 64), jnp.float32),
        ],
    )(scalar_features, W_scalar, W_ctx, W_base,
      ment.reshape(B, D_EMB), W_ent_emb, pool_sum.reshape(B, C_SPATIAL),
      W_sp_emb)

    return (lstm_input, scalar_context, baseline_feature, ee, map_skip)


# speedup vs baseline: 1.6828x; 1.0012x over previous
import jax
import jax.numpy as jnp
from jax.experimental import pallas as pl
from jax.experimental.pallas import tpu as pltpu

B, N, H, W = 16, 512, 128, 128
HW = H * W
D_ENT_IN, D_EMB = 128, 256
D_SCATTER = 32
C_SPATIAL = 64


def _entity_kernel(num_ref, ef_ref, went_ref, wproj_ref, w1_ref,
                   ee_ref, ment_ref, q_ref):
    b = pl.program_id(0)
    ef = ef_ref[0]
    ee = jax.nn.relu(jnp.dot(ef, went_ref[...],
                             preferred_element_type=jnp.float32))
    ee_ref[0] = ee
    num = num_ref[b]
    iota = jax.lax.broadcasted_iota(jnp.int32, (N, 1), 0)
    maskf = (iota < num).astype(jnp.float32)
    denom = jnp.maximum(num, 1).astype(jnp.float32)
    ment_ref[0, 0] = (ee * maskf).sum(axis=0) / denom
    proj = jax.nn.relu(jnp.dot(ee, wproj_ref[...],
                               preferred_element_type=jnp.float32)) * maskf
    q_ref[0] = jnp.dot(proj, w1_ref[...], preferred_element_type=jnp.float32)


def _spatial_kernel(idx_ref, h_ref, q_ref, w0_ref, map_ref, pool_ref):
    b = pl.program_id(0)
    h2t = h_ref[0].T
    w0v = w0_ref[...]
    for y in range(H):
        col = jax.lax.slice(h2t, (0, y), (W, y + 1))
        map_ref[y * W:(y + 1) * W, :] = col * w0v

    def body(e, carry):
        i = idx_ref[b, e]
        row = map_ref[pl.ds(i, 1), :]
        map_ref[pl.ds(i, 1), :] = row + q_ref[0, pl.ds(e, 1), :]
        return carry

    jax.lax.fori_loop(0, N, body, 0, unroll=8)
    out = jax.nn.relu(map_ref[...])
    map_ref[...] = out
    pool_ref[...] = out.sum(axis=0).reshape(1, 1, C_SPATIAL)


def _head_kernel(sf_ref, wsc_ref, wctx_ref, wbase_ref,
                 ment_ref, wee_ref, pool_ref, wsp_ref,
                 lstm_ref, ctx_ref, base_ref):
    es = jax.nn.relu(jnp.dot(sf_ref[...], wsc_ref[...],
                             preferred_element_type=jnp.float32))
    ctx_ref[...] = jax.nn.relu(jnp.dot(es, wctx_ref[...],
                                       preferred_element_type=jnp.float32))
    base_ref[...] = jax.nn.relu(jnp.dot(es, wbase_ref[...],
                                        preferred_element_type=jnp.float32))
    eent = jax.nn.relu(jnp.dot(ment_ref[...], wee_ref[...],
                               preferred_element_type=jnp.float32))
    pooled = pool_ref[...] / float(HW)
    esp = jax.nn.relu(jnp.dot(pooled, wsp_ref[...],
                              preferred_element_type=jnp.float32))
    lstm_ref[:, 0:256] = es
    lstm_ref[:, 256:512] = eent
    lstm_ref[:, 512:768] = esp


def kernel(spatial_height_map, entity_features, scalar_features, entity_x,
           entity_y, entity_num, W_scalar, W_ctx, W_base, W_ent, W_ent_emb,
           W_proj, W_spatial, W_sp_emb):
    x3 = entity_x.astype(jnp.int32).reshape(B, 1, N)
    y3 = entity_y.astype(jnp.int32).reshape(B, 1, N)
    num = entity_num.astype(jnp.int32)
    w0 = W_spatial[0:1, :]
    w1 = W_spatial[1:, :]

    ee, ment, q = pl.pallas_call(
        _entity_kernel,
        grid_spec=pltpu.PrefetchScalarGridSpec(
            num_scalar_prefetch=1,
            grid=(B,),
            in_specs=[
                pl.BlockSpec((1, N, D_ENT_IN), lambda b, *_: (b, 0, 0)),
                pl.BlockSpec((D_ENT_IN, D_EMB), lambda b, *_: (0, 0)),
                pl.BlockSpec((D_EMB, D_SCATTER), lambda b, *_: (0, 0)),
                pl.BlockSpec((D_SCATTER, C_SPATIAL), lambda b, *_: (0, 0)),
            ],
            out_specs=[
                pl.BlockSpec((1, N, D_EMB), lambda b, *_: (b, 0, 0)),
                pl.BlockSpec((1, 1, D_EMB), lambda b, *_: (b, 0, 0)),
                pl.BlockSpec((1, N, C_SPATIAL), lambda b, *_: (b, 0, 0)),
            ],
        ),
        out_shape=[
            jax.ShapeDtypeStruct((B, N, D_EMB), jnp.float32),
            jax.ShapeDtypeStruct((B, 1, D_EMB), jnp.float32),
            jax.ShapeDtypeStruct((B, N, C_SPATIAL), jnp.float32),
        ],
    )(num, entity_features, W_ent, W_proj, w1)

    fidx = (y3 * W + x3).reshape(B, N)

    map_flat, pool_sum = pl.pallas_call(
        _spatial_kernel,
        grid_spec=pltpu.PrefetchScalarGridSpec(
            num_scalar_prefetch=1,
            grid=(B,),
            in_specs=[
                pl.BlockSpec((1, H, W), lambda b, *_: (b, 0, 0)),
                pl.BlockSpec((1, N, C_SPATIAL), lambda b, *_: (b, 0, 0)),
                pl.BlockSpec((1, C_SPATIAL), lambda b, *_: (0, 0)),
            ],
            out_specs=[
                pl.BlockSpec((HW, C_SPATIAL), lambda b, *_: (b, 0)),
                pl.BlockSpec((1, 1, C_SPATIAL), lambda b, *_: (b, 0, 0)),
            ],
        ),
        out_shape=[
            jax.ShapeDtypeStruct((B * HW, C_SPATIAL), jnp.float32),
            jax.ShapeDtypeStruct((B, 1, C_SPATIAL), jnp.float32),
        ],
    )(fidx, spatial_height_map, q, w0)
    map_skip = map_flat.reshape(B, H, W, C_SPATIAL)

    lstm_input, scalar_context, baseline_feature = pl.pallas_call(
        _head_kernel,
        in_specs=[
            pl.BlockSpec((B, 256), lambda: (0, 0)),
            pl.BlockSpec((256, 256), lambda: (0, 0)),
            pl.BlockSpec((256, 128), lambda: (0, 0)),
            pl.BlockSpec((256, 64), lambda: (0, 0)),
            pl.BlockSpec((B, D_EMB), lambda: (0, 0)),
            pl.BlockSpec((D_EMB, D_EMB), lambda: (0, 0)),
            pl.BlockSpec((B, C_SPATIAL), lambda: (0, 0)),
            pl.BlockSpec((C_SPATIAL, 256), lambda: (0, 0)),
        ],
        out_specs=[
            pl.BlockSpec((B, 768), lambda: (0, 0)),
            pl.BlockSpec((B, 128), lambda: (0, 0)),
            pl.BlockSpec((B, 64), lambda: (0, 0)),
        ],
        out_shape=[
            jax.ShapeDtypeStruct((B, 768), jnp.float32),
            jax.ShapeDtypeStruct((B, 128), jnp.float32),
            jax.ShapeDtypeStruct((B, 64), jnp.float32),
        ],
    )(scalar_features, W_scalar, W_ctx, W_base,
      ment.reshape(B, D_EMB), W_ent_emb, pool_sum.reshape(B, C_SPATIAL),
      W_sp_emb)

    return (lstm_input, scalar_context, baseline_feature, ee, map_skip)
